# Initial kernel scaffold; baseline (speedup 1.0000x reference)
import jax
import jax.numpy as jnp
from jax.experimental import pallas as pl


def kernel(x, edge_index, W_pre, b_pre, W1, b1, W2, b2):
    raise NotImplementedError("write your pallas kernel here")


# speedup vs baseline: 13.1204x; 13.1204x over previous
import functools

import jax
import jax.numpy as jnp
from jax import lax
from jax.experimental import pallas as pl
from jax.experimental.pallas import tpu as pltpu
from jax.experimental.pallas import tpu_sc as plsc

_NC = 2
_NS = 16
_CH = 80


_RBIG = 640
_RSUB = 80


def _row_chunks(n, s):
    nfull = pl.cdiv(n, _RBIG) - 1
    last = n - nfull * _RBIG
    nsub = jnp.where(s < nfull, _RBIG // _RSUB, last // _RSUB)
    return s * _RBIG, nsub


@functools.lru_cache(maxsize=None)
def _make_deg(n, e):
    ept = e // (_NC * _NS)
    assert ept % _CH == 0, (n, e)
    nchunk = ept // _CH
    mesh = plsc.VectorSubcoreMesh(core_axis_name="c", subcore_axis_name="s")

    @functools.partial(
        pl.kernel,
        mesh=mesh,
        out_type=jax.ShapeDtypeStruct((_NC, n, 16), jnp.float32),
        scratch_types=[
            pltpu.VMEM((_CH,), jnp.int32),
            pltpu.VMEM((_CH, 16), jnp.float32),
            pltpu.VMEM((_RSUB, 16), jnp.float32),
            pltpu.VMEM_SHARED((n, 16), jnp.float32),
            pltpu.SemaphoreType.DMA,
        ],
    )
    def deg_kernel(dst_hbm, out_hbm, idx_v, ones_v, zbuf, acc, sem):
        c = lax.axis_index("c")
        s = lax.axis_index("s")

        def fill_ones(r, carry):
            ones_v[r, :] = jnp.ones((16,), jnp.float32)
            return carry

        lax.fori_loop(0, _CH, fill_ones, None)

        def fill_zero(r, carry):
            zbuf[r, :] = jnp.zeros((16,), jnp.float32)
            return carry

        lax.fori_loop(0, _RSUB, fill_zero, None)

        r0, nsub = _row_chunks(n, s)

        def zero_acc(k, carry):
            pltpu.sync_copy(zbuf, acc.at[pl.ds(r0 + k * _RSUB, _RSUB)])
            return carry

        lax.fori_loop(0, nsub, zero_acc, None)
        plsc.subcore_barrier()

        base = (c * _NS + s) * ept

        def chunk(i, carry):
            off = base + i * _CH
            pltpu.sync_copy(dst_hbm.at[pl.ds(off, _CH)], idx_v)
            pltpu.sync_copy(ones_v, acc.at[idx_v], add=True)
            return carry

        lax.fori_loop(0, nchunk, chunk, None)
        plsc.subcore_barrier()

        def wout(k, carry):
            rr = r0 + k * _RSUB
            pltpu.sync_copy(acc.at[pl.ds(rr, _RSUB)],
                            out_hbm.at[c, pl.ds(rr, _RSUB)])
            return carry

        lax.fori_loop(0, nsub, wout, None)

    return deg_kernel


@functools.lru_cache(maxsize=None)
def _make_agg(n, e, d):
    ept = e // (_NC * _NS)
    assert ept % _CH == 0, (n, e, d)
    nchunk = ept // _CH
    mesh = plsc.VectorSubcoreMesh(core_axis_name="c", subcore_axis_name="s")

    @functools.partial(
        pl.kernel,
        mesh=mesh,
        out_type=jax.ShapeDtypeStruct((_NC, n, d), jnp.float32),
        scratch_types=[
            pltpu.VMEM((_CH,), jnp.int32),
            pltpu.VMEM((_CH,), jnp.int32),
            pltpu.VMEM((_CH, d), jnp.float32),
            pltpu.VMEM((_RSUB, d), jnp.float32),
            pltpu.VMEM_SHARED((n, d), jnp.float32),
            pltpu.SemaphoreType.DMA,
        ],
    )
    def agg_kernel(h_hbm, src_hbm, dst_hbm, out_hbm,
                   idx_s, idx_d, rows, zbuf, acc, sem):
        c = lax.axis_index("c")
        s = lax.axis_index("s")

        def fill_zero(r, carry):
            def col(j, carry2):
                zbuf[r, pl.ds(j * 16, 16)] = jnp.zeros((16,), jnp.float32)
                return carry2
            return lax.fori_loop(0, d // 16, col, carry)

        lax.fori_loop(0, _RSUB, fill_zero, None)

        r0, nsub = _row_chunks(n, s)

        def zero_acc(k, carry):
            pltpu.sync_copy(zbuf, acc.at[pl.ds(r0 + k * _RSUB, _RSUB)])
            return carry

        lax.fori_loop(0, nsub, zero_acc, None)
        plsc.subcore_barrier()

        base = (c * _NS + s) * ept

        def chunk(i, carry):
            off = base + i * _CH
            pltpu.sync_copy(src_hbm.at[pl.ds(off, _CH)], idx_s)
            pltpu.sync_copy(dst_hbm.at[pl.ds(off, _CH)], idx_d)
            pltpu.async_copy(h_hbm.at[idx_s], rows, sem).wait()
            pltpu.sync_copy(rows, acc.at[idx_d], add=True)
            return carry

        lax.fori_loop(0, nchunk, chunk, None)
        plsc.subcore_barrier()

        def wout(k, carry):
            rr = r0 + k * _RSUB
            pltpu.sync_copy(acc.at[pl.ds(rr, _RSUB)],
                            out_hbm.at[c, pl.ds(rr, _RSUB)])
            return carry

        lax.fori_loop(0, nsub, wout, None)

    return agg_kernel


_BR = 1000


def _dinv_block(degp_ref):
    deg = degp_ref[0, :, 0:1] + degp_ref[1, :, 0:1] + 1.0
    return lax.rsqrt(deg)


def _tc1_body(x_ref, degp_ref, wp_ref, bp_ref, w1_ref, o_ref):
    dinv = _dinv_block(degp_ref)
    wc = jnp.dot(wp_ref[...], w1_ref[...], preferred_element_type=jnp.float32)
    bc = jnp.dot(bp_ref[...].reshape(1, -1), w1_ref[...],
                 preferred_element_type=jnp.float32)
    h = jnp.dot(x_ref[...], wc, preferred_element_type=jnp.float32) + bc
    o_ref[...] = h * dinv


def _tc2_body(a_ref, hs1_ref, degp_ref, b1_ref, w2_ref, o_ref):
    dinv = _dinv_block(degp_ref)
    t = (a_ref[0] + a_ref[1] + hs1_ref[...]) * dinv + b1_ref[...]
    t = jnp.maximum(t, 0.0)
    o_ref[...] = jnp.dot(t, w2_ref[...],
                         preferred_element_type=jnp.float32) * dinv


def _tc3_body(a_ref, hs2_ref, degp_ref, b2_ref, o_ref):
    dinv = _dinv_block(degp_ref)
    y = (a_ref[0] + a_ref[1] + hs2_ref[...]) * dinv + b2_ref[...]
    nrm = jnp.sqrt(jnp.sum(y * y, axis=1, keepdims=True))
    o_ref[...] = y / jnp.maximum(nrm, 1e-12)


def _row_spec(d):
    return pl.BlockSpec((_BR, d), lambda i: (i, 0))


def _part_spec(d):
    return pl.BlockSpec((_NC, _BR, d), lambda i: (0, i, 0))


def _full_spec(shape):
    nd = len(shape)
    return pl.BlockSpec(shape, lambda i: (0,) * nd)


@functools.lru_cache(maxsize=None)
def _make_tc1(n, d):
    return pl.pallas_call(
        _tc1_body,
        grid=(n // _BR,),
        in_specs=[_row_spec(d), _part_spec(16), _full_spec((d, d)),
                  _full_spec((d,)), _full_spec((d, d))],
        out_specs=_row_spec(d),
        out_shape=jax.ShapeDtypeStruct((n, d), jnp.float32),
    )


@functools.lru_cache(maxsize=None)
def _make_tc2(n, d):
    return pl.pallas_call(
        _tc2_body,
        grid=(n // _BR,),
        in_specs=[_part_spec(d), _row_spec(d), _part_spec(16),
                  _full_spec((d,)), _full_spec((d, d))],
        out_specs=_row_spec(d),
        out_shape=jax.ShapeDtypeStruct((n, d), jnp.float32),
    )


@functools.lru_cache(maxsize=None)
def _make_tc3(n, d):
    return pl.pallas_call(
        _tc3_body,
        grid=(n // _BR,),
        in_specs=[_part_spec(d), _row_spec(d), _part_spec(16),
                  _full_spec((d,))],
        out_specs=_row_spec(d),
        out_shape=jax.ShapeDtypeStruct((n, d), jnp.float32),
    )


def kernel(x, edge_index, W_pre, b_pre, W1, b1, W2, b2):
    n, d = x.shape
    e = edge_index.shape[1]
    ei = edge_index.astype(jnp.int32)
    src, dst = ei[0], ei[1]

    degp = _make_deg(n, e)(dst)
    hs1 = _make_tc1(n, d)(x, degp, W_pre, b_pre, W1)
    agg1 = _make_agg(n, e, d)(hs1, src, dst)
    hs2 = _make_tc2(n, d)(agg1, hs1, degp, b1, W2)
    agg2 = _make_agg(n, e, d)(hs2, src, dst)
    return _make_tc3(n, d)(agg2, hs2, degp, b2)
